# baseline (device time: 23805 ns/iter reference)
import jax
import jax.numpy as jnp
from jax import lax
from jax.experimental import pallas as pl
from jax.experimental.pallas import tpu as pltpu

N_DEV = 4
E_PER = 4
CAP = 51
N_TOK = 1024
D_MODEL = 256
D_FF = 512
M = N_TOK // N_DEV


def kernel(x, router_W, route_idx, expert_W):
    del router_W

    def body(x_ref, ridx_ref, w_ref, out_ref,
             partial, send_buf, comm, send_sems, recv_sems):
        my = lax.axis_index("i")

        barrier = pltpu.get_barrier_semaphore()
        for o in range(1, N_DEV):
            pl.semaphore_signal(
                barrier, inc=1,
                device_id=((my + o) % N_DEV,),
                device_id_type=pl.DeviceIdType.MESH,
            )
        pl.semaphore_wait(barrier, N_DEV - 1)

        xv = x_ref[:, :]
        ridx = ridx_ref[:, :]

        lane = lax.broadcasted_iota(jnp.int32, (1, 128), 1)
        eid = jnp.where(lane < E_PER, my * E_PER + lane, -7)
        onehot = (ridx == eid).astype(jnp.float32)

        row = lax.broadcasted_iota(jnp.int32, (N_TOK, N_TOK), 0)
        col = lax.broadcasted_iota(jnp.int32, (N_TOK, N_TOK), 1)
        ltri = (col < row).astype(jnp.float32)
        rank = jnp.dot(ltri, onehot, preferred_element_type=jnp.float32)
        keep = onehot * (rank < CAP - 0.5).astype(jnp.float32)

        acc = jnp.zeros((N_TOK, D_FF), jnp.float32)
        for k in range(E_PER):
            xm = xv * keep[:, k:k + 1]
            acc = acc + jnp.dot(xm, w_ref[k], preferred_element_type=jnp.float32)
        partial[:, :] = acc

        rdmas = []
        for o in range(1, N_DEV):
            tgt = (my + o) % N_DEV
            send_buf[o - 1, :, :] = partial[pl.ds(tgt * M, M), :]
            rdma = pltpu.make_async_remote_copy(
                src_ref=send_buf.at[o - 1],
                dst_ref=comm.at[N_DEV - 1 - o],
                send_sem=send_sems.at[o - 1],
                recv_sem=recv_sems.at[N_DEV - 1 - o],
                device_id=(tgt,),
                device_id_type=pl.DeviceIdType.MESH,
            )
            rdma.start()
            rdmas.append(rdma)
        for rdma in rdmas:
            rdma.wait()

        out_ref[:, :] = (
            partial[pl.ds(my * M, M), :]
            + comm[0] + comm[1] + comm[2]
        )

    return pl.pallas_call(
        body,
        out_shape=jax.ShapeDtypeStruct((M, D_FF), jnp.float32),
        in_specs=[
            pl.BlockSpec(memory_space=pltpu.VMEM),
            pl.BlockSpec(memory_space=pltpu.VMEM),
            pl.BlockSpec(memory_space=pltpu.VMEM),
        ],
        out_specs=pl.BlockSpec(memory_space=pltpu.VMEM),
        scratch_shapes=[
            pltpu.VMEM((N_TOK, D_FF), jnp.float32),
            pltpu.VMEM((N_DEV - 1, M, D_FF), jnp.float32),
            pltpu.VMEM((N_DEV - 1, M, D_FF), jnp.float32),
            pltpu.SemaphoreType.DMA((N_DEV - 1,)),
            pltpu.SemaphoreType.DMA((N_DEV - 1,)),
        ],
        compiler_params=pltpu.CompilerParams(collective_id=0),
    )(x, route_idx, expert_W)


# device time: 17992 ns/iter; 1.3231x vs baseline; 1.3231x over previous
import jax
import jax.numpy as jnp
from jax import lax
from jax.experimental import pallas as pl
from jax.experimental.pallas import tpu as pltpu

N_DEV = 4
E_PER = 4
CAP = 51
N_TOK = 1024
D_MODEL = 256
D_FF = 512
M = N_TOK // N_DEV


def kernel(x, router_W, route_idx, expert_W):
    del router_W

    def body(x_ref, ridx_ref, w_ref, out_ref,
             keep_ref, send_buf, comm, send_sems, recv_sems):
        my = lax.axis_index("i")

        barrier = pltpu.get_barrier_semaphore()
        for o in range(1, N_DEV):
            pl.semaphore_signal(
                barrier, inc=1,
                device_id=((my + o) % N_DEV,),
                device_id_type=pl.DeviceIdType.MESH,
            )
        pl.semaphore_wait(barrier, N_DEV - 1)

        ridx = ridx_ref[:, :]
        lane = lax.broadcasted_iota(jnp.int32, (1, 128), 1)
        eid = jnp.where(lane < E_PER, my * E_PER + lane, -7)
        onehot = (ridx == eid).astype(jnp.float32)

        row = lax.broadcasted_iota(jnp.int32, (N_TOK, N_TOK), 0)
        col = lax.broadcasted_iota(jnp.int32, (N_TOK, N_TOK), 1)
        ltri = (col < row).astype(jnp.float32)
        rank = jnp.dot(ltri, onehot, preferred_element_type=jnp.float32)
        keep_ref[:, :] = onehot * (rank < CAP - 0.5).astype(jnp.float32)

        def block_partial(off):
            xb = x_ref[pl.ds(off, M), :]
            kb = keep_ref[pl.ds(off, M), :]
            acc = jnp.zeros((M, D_FF), jnp.float32)
            for k in range(E_PER):
                acc = acc + jnp.dot(xb * kb[:, k:k + 1], w_ref[k],
                                    preferred_element_type=jnp.float32)
            return acc

        rdmas = []
        for o in range(1, N_DEV):
            tgt = (my + o) % N_DEV
            send_buf[o - 1, :, :] = block_partial(tgt * M).astype(jnp.bfloat16)
            rdma = pltpu.make_async_remote_copy(
                src_ref=send_buf.at[o - 1],
                dst_ref=comm.at[N_DEV - 1 - o],
                send_sem=send_sems.at[o - 1],
                recv_sem=recv_sems.at[N_DEV - 1 - o],
                device_id=(tgt,),
                device_id_type=pl.DeviceIdType.MESH,
            )
            rdma.start()
            rdmas.append(rdma)

        own = block_partial(my * M)
        for rdma in rdmas:
            rdma.wait()

        out_ref[:, :] = (
            own
            + comm[0].astype(jnp.float32)
            + comm[1].astype(jnp.float32)
            + comm[2].astype(jnp.float32)
        )

    return pl.pallas_call(
        body,
        out_shape=jax.ShapeDtypeStruct((M, D_FF), jnp.float32),
        in_specs=[
            pl.BlockSpec(memory_space=pltpu.VMEM),
            pl.BlockSpec(memory_space=pltpu.VMEM),
            pl.BlockSpec(memory_space=pltpu.VMEM),
        ],
        out_specs=pl.BlockSpec(memory_space=pltpu.VMEM),
        scratch_shapes=[
            pltpu.VMEM((N_TOK, 128), jnp.float32),
            pltpu.VMEM((N_DEV - 1, M, D_FF), jnp.bfloat16),
            pltpu.VMEM((N_DEV - 1, M, D_FF), jnp.bfloat16),
            pltpu.SemaphoreType.DMA((N_DEV - 1,)),
            pltpu.SemaphoreType.DMA((N_DEV - 1,)),
        ],
        compiler_params=pltpu.CompilerParams(collective_id=0),
    )(x, route_idx, expert_W)
